# Initial kernel scaffold; baseline (speedup 1.0000x reference)
import jax
import jax.numpy as jnp
from jax.experimental import pallas as pl


def kernel(x, edge_index, W1, b1, W2, b2):
    raise NotImplementedError("write your pallas kernel here")


# speedup vs baseline: 19.8045x; 19.8045x over previous
import functools

import jax
import jax.numpy as jnp
from jax import lax
from jax.experimental import pallas as pl
from jax.experimental.pallas import tpu as pltpu
from jax.experimental.pallas import tpu_sc as plsc

N = 10000
E = 320000
D = 128
N2 = 10240

NC = 2
NS = 16
NW = NC * NS
EDGES_PER_TILE = E // NW
K = 50
NWIN = EDGES_PER_TILE // K
GROUP = 4
NGROUP = NWIN // GROUP
ROWS_PER_TILE = N2 // NS
DEG_PER_TILE = N2 // NS

_mesh = plsc.VectorSubcoreMesh(core_axis_name="c", subcore_axis_name="s")


@functools.partial(
    pl.kernel,
    out_type=jax.ShapeDtypeStruct((NC, 1, N2), jnp.float32),
    mesh=_mesh,
    scratch_types=[
        pltpu.VMEM_SHARED((N2,), jnp.float32),
        pltpu.VMEM((NWIN, 2, K), jnp.int32),
        pltpu.VMEM((DEG_PER_TILE,), jnp.float32),
        pltpu.VMEM((64,), jnp.float32),
    ],
)
def _deg_kernel(es_hbm, out_hbm, deg_acc, idx_v, zero_v, ones_v):
    c = lax.axis_index("c")
    s = lax.axis_index("s")
    wid = c * NS + s
    for i in range(DEG_PER_TILE // 16):
        zero_v[pl.ds(i * 16, 16)] = jnp.zeros((16,), jnp.float32)
    for i in range(4):
        ones_v[pl.ds(i * 16, 16)] = jnp.ones((16,), jnp.float32)
    sl = pl.ds(s * DEG_PER_TILE, DEG_PER_TILE)
    pltpu.sync_copy(zero_v, deg_acc.at[sl])
    pltpu.sync_copy(es_hbm.at[wid], idx_v)
    plsc.subcore_barrier()

    def body(w, carry):
        pltpu.sync_copy(ones_v.at[pl.ds(0, K)], deg_acc.at[idx_v.at[w, 1]],
                        add=True)
        return carry

    lax.fori_loop(0, NWIN, body, 0)
    plsc.subcore_barrier()
    pltpu.sync_copy(deg_acc.at[sl], out_hbm.at[c, 0, sl])


@functools.partial(
    pl.kernel,
    out_type=jax.ShapeDtypeStruct((NC, N2, D), jnp.float32),
    mesh=_mesh,
    scratch_types=[
        pltpu.VMEM_SHARED((N2, D), jnp.float32),
        pltpu.VMEM((GROUP, 2, K), jnp.int32),
        pltpu.VMEM((GROUP, K, D), jnp.float32),
        pltpu.SemaphoreType.DMA,
        pltpu.SemaphoreType.DMA,
        pltpu.SemaphoreType.DMA,
        pltpu.SemaphoreType.DMA,
    ],
)
def _agg_kernel(g_hbm, es_hbm, out_hbm, acc, idx_v, rows_v,
                sem0, sem1, sem2, sem3):
    sems = (sem0, sem1, sem2, sem3)
    c = lax.axis_index("c")
    s = lax.axis_index("s")
    wid = c * NS + s
    rsl = pl.ds(s * ROWS_PER_TILE, ROWS_PER_TILE)
    pltpu.sync_copy(g_hbm.at[rsl], acc.at[rsl])
    plsc.subcore_barrier()

    def group_body(gi, carry):
        pltpu.sync_copy(es_hbm.at[wid, pl.ds(gi * GROUP, GROUP)], idx_v)
        handles = []
        for b in range(GROUP):
            handles.append(
                pltpu.async_copy(g_hbm.at[idx_v.at[b, 0]], rows_v.at[b],
                                 sems[b]))
        for b in range(GROUP):
            handles[b].wait()
            pltpu.sync_copy(rows_v.at[b], acc.at[idx_v.at[b, 1]], add=True)
        return carry

    lax.fori_loop(0, NGROUP, group_body, 0)
    plsc.subcore_barrier()
    pltpu.sync_copy(acc.at[rsl], out_hbm.at[c, rsl])


ROWS_BLK = 1000
ROWS_BLK2 = 1024


def _mm1_body(x_ref, w_ref, d0_ref, d1_ref, g_ref, dinv_ref):
    dinv = lax.rsqrt(d0_ref[...] + d1_ref[...] + 1.0)
    dinv_ref[...] = dinv
    g_ref[...] = jnp.dot(x_ref[...], w_ref[...],
                         preferred_element_type=jnp.float32) * dinv


def _mid_body(p0_ref, p1_ref, g_ref, dinv_ref, b_ref, w_ref, o_ref):
    h = (p0_ref[...] + p1_ref[...] - g_ref[...]) * dinv_ref[...] + b_ref[...]
    o_ref[...] = jnp.dot(h, w_ref[...],
                         preferred_element_type=jnp.float32) * dinv_ref[...]


def _fin_body(p0_ref, p1_ref, g_ref, dinv_ref, b_ref, o_ref):
    o_ref[...] = ((p0_ref[...] + p1_ref[...] - g_ref[...]) * dinv_ref[...]
                  + b_ref[...])


def _row_specs(blk):
    return (pl.BlockSpec((blk, D), lambda i: (i, 0)),
            pl.BlockSpec((blk, 1), lambda i: (i, 0)))


_w_spec = pl.BlockSpec((D, D), lambda i: (0, 0))
_b_spec = pl.BlockSpec((1, D), lambda i: (0, 0))


def kernel(x, edge_index, W1, b1, W2, b2):
    es = jnp.stack([edge_index[0].reshape(NW, NWIN, K),
                    edge_index[1].reshape(NW, NWIN, K)], axis=2)
    b1r = b1.reshape(1, D)
    b2r = b2.reshape(1, D)

    degp = _deg_kernel(es)
    d0 = degp[0, 0, :, None]
    d1 = degp[1, 0, :, None]

    row_spec, col_spec = _row_specs(ROWS_BLK)
    row_spec2, col_spec2 = _row_specs(ROWS_BLK2)

    g1, dinv = pl.pallas_call(
        _mm1_body,
        grid=(N // ROWS_BLK,),
        in_specs=[row_spec, _w_spec, col_spec, col_spec],
        out_specs=(row_spec, col_spec),
        out_shape=(jax.ShapeDtypeStruct((N2, D), jnp.float32),
                   jax.ShapeDtypeStruct((N2, 1), jnp.float32)),
    )(x, W1, d0, d1)

    p = _agg_kernel(g1, es)

    g2 = pl.pallas_call(
        _mid_body,
        grid=(N2 // ROWS_BLK2,),
        in_specs=[row_spec2, row_spec2, row_spec2, col_spec2, _b_spec,
                  _w_spec],
        out_specs=row_spec2,
        out_shape=jax.ShapeDtypeStruct((N2, D), jnp.float32),
    )(p[0], p[1], g1, dinv, b1r, W2)

    q = _agg_kernel(g2, es)

    out = pl.pallas_call(
        _fin_body,
        grid=(N // ROWS_BLK,),
        in_specs=[row_spec, row_spec, row_spec, col_spec, _b_spec],
        out_specs=row_spec,
        out_shape=jax.ShapeDtypeStruct((N, D), jnp.float32),
    )(q[0], q[1], g2, dinv, b2r)

    return out


# speedup vs baseline: 25.6229x; 1.2938x over previous
import functools

import jax
import jax.numpy as jnp
from jax import lax
from jax.experimental import pallas as pl
from jax.experimental.pallas import tpu as pltpu
from jax.experimental.pallas import tpu_sc as plsc

N = 10000
E = 320000
D = 128
N2 = 10240

NC = 2
NS = 16
NW = NC * NS
EDGES_PER_TILE = E // NW
K = 50
NWIN = EDGES_PER_TILE // K
GROUP = 2
NGROUP = NWIN // GROUP
ROWS_PER_TILE = N2 // NS
DEG_PER_TILE = N2 // NS

_mesh = plsc.VectorSubcoreMesh(core_axis_name="c", subcore_axis_name="s")


@functools.partial(
    pl.kernel,
    out_type=jax.ShapeDtypeStruct((NC, 1, N2), jnp.float32),
    mesh=_mesh,
    scratch_types=[
        pltpu.VMEM_SHARED((N2,), jnp.float32),
        pltpu.VMEM((NWIN, 2, K), jnp.int32),
        pltpu.VMEM((DEG_PER_TILE,), jnp.float32),
        pltpu.VMEM((64,), jnp.float32),
    ],
)
def _deg_kernel(es_hbm, out_hbm, deg_acc, idx_v, zero_v, ones_v):
    c = lax.axis_index("c")
    s = lax.axis_index("s")
    wid = c * NS + s
    for i in range(DEG_PER_TILE // 16):
        zero_v[pl.ds(i * 16, 16)] = jnp.zeros((16,), jnp.float32)
    for i in range(4):
        ones_v[pl.ds(i * 16, 16)] = jnp.ones((16,), jnp.float32)
    sl = pl.ds(s * DEG_PER_TILE, DEG_PER_TILE)
    pltpu.sync_copy(zero_v, deg_acc.at[sl])
    pltpu.sync_copy(es_hbm.at[wid], idx_v)
    plsc.subcore_barrier()

    def body(w, carry):
        pltpu.sync_copy(ones_v.at[pl.ds(0, K)], deg_acc.at[idx_v.at[w, 1]],
                        add=True)
        return carry

    lax.fori_loop(0, NWIN, body, 0)
    plsc.subcore_barrier()
    pltpu.sync_copy(deg_acc.at[sl], out_hbm.at[c, 0, sl])


@functools.partial(
    pl.kernel,
    out_type=jax.ShapeDtypeStruct((NC, N2, D), jnp.float32),
    mesh=_mesh,
    scratch_types=[
        pltpu.VMEM_SHARED((N2, D), jnp.float32),
        pltpu.VMEM((GROUP, 2, K), jnp.int32),
        pltpu.VMEM((GROUP, 2, K), jnp.int32),
        pltpu.VMEM((GROUP, K, D), jnp.float32),
        pltpu.VMEM((GROUP, K, D), jnp.float32),
        pltpu.SemaphoreType.DMA,
        pltpu.SemaphoreType.DMA,
        pltpu.SemaphoreType.DMA,
        pltpu.SemaphoreType.DMA,
    ],
)
def _agg_kernel(g_hbm, es_hbm, out_hbm, acc, idx_a, idx_b, rows_a, rows_b,
                sem0, sem1, sem2, sem3):
    sets = ((idx_a, rows_a, (sem0, sem1)), (idx_b, rows_b, (sem2, sem3)))
    c = lax.axis_index("c")
    s = lax.axis_index("s")
    wid = c * NS + s
    rsl = pl.ds(s * ROWS_PER_TILE, ROWS_PER_TILE)
    pltpu.sync_copy(g_hbm.at[rsl], acc.at[rsl])
    plsc.subcore_barrier()

    def fire(gi, st):
        idx_v, rows_v, sems = st
        pltpu.sync_copy(es_hbm.at[wid, pl.ds(gi * GROUP, GROUP)], idx_v)
        for b in range(GROUP):
            pltpu.async_copy(g_hbm.at[idx_v.at[b, 0]], rows_v.at[b], sems[b])

    def drain(st):
        idx_v, rows_v, sems = st
        for b in range(GROUP):
            pltpu.make_async_copy(g_hbm.at[idx_v.at[b, 0]], rows_v.at[b],
                                  sems[b]).wait()
            pltpu.sync_copy(rows_v.at[b], acc.at[idx_v.at[b, 1]], add=True)

    fire(0, sets[0])

    def pair_body(pi, carry):
        fire(2 * pi + 1, sets[1])
        drain(sets[0])
        fire(2 * pi + 2, sets[0])
        drain(sets[1])
        return carry

    lax.fori_loop(0, NGROUP // 2 - 1, pair_body, 0)
    fire(NGROUP - 1, sets[1])
    drain(sets[0])
    drain(sets[1])
    plsc.subcore_barrier()
    pltpu.sync_copy(acc.at[rsl], out_hbm.at[c, rsl])


ROWS_BLK = 1000
ROWS_BLK2 = 1024


def _mm1_body(x_ref, w_ref, d0_ref, d1_ref, g_ref, dinv_ref):
    dinv = lax.rsqrt(d0_ref[...] + d1_ref[...] + 1.0)
    dinv_ref[...] = dinv
    g_ref[...] = jnp.dot(x_ref[...], w_ref[...],
                         preferred_element_type=jnp.float32) * dinv


def _mid_body(p0_ref, p1_ref, g_ref, dinv_ref, b_ref, w_ref, o_ref):
    h = (p0_ref[0] + p1_ref[0] - g_ref[...]) * dinv_ref[...] + b_ref[...]
    o_ref[...] = jnp.dot(h, w_ref[...],
                         preferred_element_type=jnp.float32) * dinv_ref[...]


def _fin_body(p0_ref, p1_ref, g_ref, dinv_ref, b_ref, o_ref):
    o_ref[...] = ((p0_ref[0] + p1_ref[0] - g_ref[...]) * dinv_ref[...]
                  + b_ref[...])


def _row_specs(blk):
    return (pl.BlockSpec((blk, D), lambda i: (i, 0)),
            pl.BlockSpec((blk, 1), lambda i: (i, 0)))


_w_spec = pl.BlockSpec((D, D), lambda i: (0, 0))
_b_spec = pl.BlockSpec((1, D), lambda i: (0, 0))


def kernel(x, edge_index, W1, b1, W2, b2):
    es = jnp.stack([edge_index[0].reshape(NW, NWIN, K),
                    edge_index[1].reshape(NW, NWIN, K)], axis=2)
    b1r = b1.reshape(1, D)
    b2r = b2.reshape(1, D)

    degp = _deg_kernel(es)
    d0 = degp[0, 0, :, None]
    d1 = degp[1, 0, :, None]

    row_spec, col_spec = _row_specs(ROWS_BLK)
    row_spec2, col_spec2 = _row_specs(ROWS_BLK2)

    g1, dinv = pl.pallas_call(
        _mm1_body,
        grid=(N // ROWS_BLK,),
        in_specs=[row_spec, _w_spec, col_spec, col_spec],
        out_specs=(row_spec, col_spec),
        out_shape=(jax.ShapeDtypeStruct((N2, D), jnp.float32),
                   jax.ShapeDtypeStruct((N2, 1), jnp.float32)),
    )(x, W1, d0, d1)

    p = _agg_kernel(g1, es)

    part0_spec2 = pl.BlockSpec((1, ROWS_BLK2, D), lambda i: (0, i, 0))
    part1_spec2 = pl.BlockSpec((1, ROWS_BLK2, D), lambda i: (1, i, 0))
    g2 = pl.pallas_call(
        _mid_body,
        grid=(N2 // ROWS_BLK2,),
        in_specs=[part0_spec2, part1_spec2, row_spec2, col_spec2, _b_spec,
                  _w_spec],
        out_specs=row_spec2,
        out_shape=jax.ShapeDtypeStruct((N2, D), jnp.float32),
    )(p, p, g1, dinv, b1r, W2)

    q = _agg_kernel(g2, es)

    part0_spec = pl.BlockSpec((1, ROWS_BLK, D), lambda i: (0, i, 0))
    part1_spec = pl.BlockSpec((1, ROWS_BLK, D), lambda i: (1, i, 0))
    out = pl.pallas_call(
        _fin_body,
        grid=(N // ROWS_BLK,),
        in_specs=[part0_spec, part1_spec, row_spec, col_spec, _b_spec],
        out_specs=row_spec,
        out_shape=jax.ShapeDtypeStruct((N, D), jnp.float32),
    )(q, q, g2, dinv, b2r)

    return out


# speedup vs baseline: 31.1599x; 1.2161x over previous
import functools

import jax
import jax.numpy as jnp
from jax import lax
from jax.experimental import pallas as pl
from jax.experimental.pallas import tpu as pltpu
from jax.experimental.pallas import tpu_sc as plsc

N = 10000
E = 320000
D = 128
N2 = 10240

NC = 2
NS = 16
NW = NC * NS
EDGES_PER_TILE = E // NW
K = 50
NWIN = EDGES_PER_TILE // K
GROUP = 2
NGROUP = NWIN // GROUP
ROWS_PER_TILE = N2 // NS
DEG_PER_TILE = N2 // NS

_mesh = plsc.VectorSubcoreMesh(core_axis_name="c", subcore_axis_name="s")


KD = 128
NWIN_D = 79
EPT_PAD = NWIN_D * KD


@functools.partial(
    pl.kernel,
    out_type=jax.ShapeDtypeStruct((NC, 1, N2), jnp.float32),
    mesh=_mesh,
    scratch_types=[
        pltpu.VMEM_SHARED((N2,), jnp.float32),
        pltpu.VMEM((NWIN_D, 1, KD), jnp.int32),
        pltpu.VMEM((DEG_PER_TILE,), jnp.float32),
        pltpu.VMEM((KD,), jnp.float32),
    ],
)
def _deg_kernel(dst_hbm, out_hbm, deg_acc, idx_v, zero_v, ones_v):
    c = lax.axis_index("c")
    s = lax.axis_index("s")
    wid = c * NS + s
    for i in range(DEG_PER_TILE // 16):
        zero_v[pl.ds(i * 16, 16)] = jnp.zeros((16,), jnp.float32)
    for i in range(KD // 16):
        ones_v[pl.ds(i * 16, 16)] = jnp.ones((16,), jnp.float32)
    sl = pl.ds(s * DEG_PER_TILE, DEG_PER_TILE)
    pltpu.sync_copy(zero_v, deg_acc.at[sl])
    pltpu.sync_copy(dst_hbm.at[wid], idx_v)
    plsc.subcore_barrier()

    def body(w, carry):
        pltpu.sync_copy(ones_v, deg_acc.at[idx_v.at[w, 0]], add=True)
        return carry

    lax.fori_loop(0, NWIN_D, body, 0)
    plsc.subcore_barrier()
    pltpu.sync_copy(deg_acc.at[sl], out_hbm.at[c, 0, sl])


@functools.partial(
    pl.kernel,
    out_type=jax.ShapeDtypeStruct((NC, N2, D), jnp.float32),
    mesh=_mesh,
    scratch_types=[
        pltpu.VMEM_SHARED((N2, D), jnp.float32),
        pltpu.VMEM((GROUP, 2, K), jnp.int32),
        pltpu.VMEM((GROUP, 2, K), jnp.int32),
        pltpu.VMEM((GROUP, 2, K), jnp.int32),
        pltpu.VMEM((GROUP, 2, K), jnp.int32),
        pltpu.VMEM((GROUP, K, D), jnp.float32),
        pltpu.VMEM((GROUP, K, D), jnp.float32),
        pltpu.SemaphoreType.DMA,
        pltpu.SemaphoreType.DMA,
        pltpu.SemaphoreType.DMA,
        pltpu.SemaphoreType.DMA,
        pltpu.SemaphoreType.DMA,
        pltpu.SemaphoreType.DMA,
        pltpu.SemaphoreType.DMA,
        pltpu.SemaphoreType.DMA,
    ],
)
def _agg_kernel(g_hbm, es_hbm, out_hbm, acc, i0, i1, i2, i3, rows_a, rows_b,
                is0, is1, is2, is3, gs0, gs1, gs2, gs3):
    idx = (i0, i1, i2, i3)
    isem = (is0, is1, is2, is3)
    rows = (rows_a, rows_b)
    gsem = ((gs0, gs1), (gs2, gs3))
    c = lax.axis_index("c")
    s = lax.axis_index("s")
    wid = c * NS + s
    rsl = pl.ds(s * ROWS_PER_TILE, ROWS_PER_TILE)
    pltpu.sync_copy(g_hbm.at[rsl], acc.at[rsl])
    plsc.subcore_barrier()

    def fire_idx(g, j):
        pltpu.async_copy(es_hbm.at[wid, pl.ds(g * GROUP, GROUP)], idx[j],
                         isem[j])

    def fire_gathers(g, j):
        pltpu.make_async_copy(es_hbm.at[wid, pl.ds(g * GROUP, GROUP)],
                              idx[j], isem[j]).wait()
        for b in range(GROUP):
            pltpu.async_copy(g_hbm.at[idx[j].at[b, 0]], rows[j % 2].at[b],
                             gsem[j % 2][b])

    def drain(g, j):
        for b in range(GROUP):
            pltpu.make_async_copy(g_hbm.at[idx[j].at[b, 0]],
                                  rows[j % 2].at[b], gsem[j % 2][b]).wait()
            pltpu.sync_copy(rows[j % 2].at[b], acc.at[idx[j].at[b, 1]],
                            add=True)

    fire_idx(0, 0)
    fire_idx(1, 1)
    fire_idx(2, 2)
    fire_gathers(0, 0)

    def quad_body(qi, carry):
        g0 = 4 * qi
        for j in range(4):
            fire_gathers(g0 + j + 1, (j + 1) % 4)
            drain(g0 + j, j)
            fire_idx(g0 + j + 3, (j + 3) % 4)
        return carry

    lax.fori_loop(0, NGROUP // 4 - 1, quad_body, 0)
    fire_gathers(NGROUP - 3, 1)
    drain(NGROUP - 4, 0)
    fire_idx(NGROUP - 1, 3)
    fire_gathers(NGROUP - 2, 2)
    drain(NGROUP - 3, 1)
    fire_gathers(NGROUP - 1, 3)
    drain(NGROUP - 2, 2)
    drain(NGROUP - 1, 3)
    plsc.subcore_barrier()
    pltpu.sync_copy(acc.at[rsl], out_hbm.at[c, rsl])


ROWS_BLK = 1000
ROWS_BLK2 = 1024


def _mm1_body(x_ref, w_ref, d0_ref, d1_ref, g_ref, dinv_ref):
    dinv = lax.rsqrt(d0_ref[...] + d1_ref[...] + 1.0)
    dinv_ref[...] = dinv
    g_ref[...] = jnp.dot(x_ref[...], w_ref[...],
                         preferred_element_type=jnp.float32) * dinv


def _mid_body(p0_ref, p1_ref, g_ref, dinv_ref, b_ref, w_ref, o_ref):
    h = (p0_ref[0] + p1_ref[0] - g_ref[...]) * dinv_ref[...] + b_ref[...]
    o_ref[...] = jnp.dot(h, w_ref[...],
                         preferred_element_type=jnp.float32) * dinv_ref[...]


def _fin_body(p0_ref, p1_ref, g_ref, dinv_ref, b_ref, o_ref):
    o_ref[...] = ((p0_ref[0] + p1_ref[0] - g_ref[...]) * dinv_ref[...]
                  + b_ref[...])


def _row_specs(blk):
    return (pl.BlockSpec((blk, D), lambda i: (i, 0)),
            pl.BlockSpec((blk, 1), lambda i: (i, 0)))


_w_spec = pl.BlockSpec((D, D), lambda i: (0, 0))
_b_spec = pl.BlockSpec((1, D), lambda i: (0, 0))


def kernel(x, edge_index, W1, b1, W2, b2):
    es = jnp.stack([edge_index[0].reshape(NW, NWIN, K),
                    edge_index[1].reshape(NW, NWIN, K)], axis=2)
    dstp = jnp.pad(edge_index[1].reshape(NW, EDGES_PER_TILE),
                   ((0, 0), (0, EPT_PAD - EDGES_PER_TILE)),
                   constant_values=N2 - 1).reshape(NW, NWIN_D, 1, KD)
    b1r = b1.reshape(1, D)
    b2r = b2.reshape(1, D)

    degp = _deg_kernel(dstp)
    d0 = degp[0, 0, :, None]
    d1 = degp[1, 0, :, None]

    row_spec, col_spec = _row_specs(ROWS_BLK)
    row_spec2, col_spec2 = _row_specs(ROWS_BLK2)

    g1, dinv = pl.pallas_call(
        _mm1_body,
        grid=(N // ROWS_BLK,),
        in_specs=[row_spec, _w_spec, col_spec, col_spec],
        out_specs=(row_spec, col_spec),
        out_shape=(jax.ShapeDtypeStruct((N2, D), jnp.float32),
                   jax.ShapeDtypeStruct((N2, 1), jnp.float32)),
    )(x, W1, d0, d1)

    p = _agg_kernel(g1, es)

    part0_spec2 = pl.BlockSpec((1, ROWS_BLK2, D), lambda i: (0, i, 0))
    part1_spec2 = pl.BlockSpec((1, ROWS_BLK2, D), lambda i: (1, i, 0))
    g2 = pl.pallas_call(
        _mid_body,
        grid=(N2 // ROWS_BLK2,),
        in_specs=[part0_spec2, part1_spec2, row_spec2, col_spec2, _b_spec,
                  _w_spec],
        out_specs=row_spec2,
        out_shape=jax.ShapeDtypeStruct((N2, D), jnp.float32),
    )(p, p, g1, dinv, b1r, W2)

    q = _agg_kernel(g2, es)

    part0_spec = pl.BlockSpec((1, ROWS_BLK, D), lambda i: (0, i, 0))
    part1_spec = pl.BlockSpec((1, ROWS_BLK, D), lambda i: (1, i, 0))
    out = pl.pallas_call(
        _fin_body,
        grid=(N // ROWS_BLK,),
        in_specs=[part0_spec, part1_spec, row_spec, col_spec, _b_spec],
        out_specs=row_spec,
        out_shape=jax.ShapeDtypeStruct((N, D), jnp.float32),
    )(q, q, g2, dinv, b2r)

    return out


# speedup vs baseline: 31.5189x; 1.0115x over previous
import functools

import jax
import jax.numpy as jnp
from jax import lax
from jax.experimental import pallas as pl
from jax.experimental.pallas import tpu as pltpu
from jax.experimental.pallas import tpu_sc as plsc

N = 10000
E = 320000
D = 128
N2 = 10240

NC = 2
NS = 16
NW = NC * NS
EDGES_PER_TILE = E // NW
K = 50
NWIN = EDGES_PER_TILE // K
GROUP = 2
NGROUP = NWIN // GROUP
ROWS_PER_TILE = N2 // NS
DEG_PER_TILE = N2 // NS

_mesh = plsc.VectorSubcoreMesh(core_axis_name="c", subcore_axis_name="s")


KD = 128
NWIN_D = 79
EPT_PAD = NWIN_D * KD


@functools.partial(
    pl.kernel,
    out_type=jax.ShapeDtypeStruct((NC, 1, N2), jnp.float32),
    mesh=_mesh,
    scratch_types=[
        pltpu.VMEM_SHARED((N2,), jnp.float32),
        pltpu.VMEM((NWIN_D, 1, KD), jnp.int32),
        pltpu.VMEM((DEG_PER_TILE,), jnp.float32),
        pltpu.VMEM((KD,), jnp.float32),
    ],
)
def _deg_kernel(dst_hbm, out_hbm, deg_acc, idx_v, zero_v, ones_v):
    c = lax.axis_index("c")
    s = lax.axis_index("s")
    wid = c * NS + s
    for i in range(DEG_PER_TILE // 16):
        zero_v[pl.ds(i * 16, 16)] = jnp.zeros((16,), jnp.float32)
    for i in range(KD // 16):
        ones_v[pl.ds(i * 16, 16)] = jnp.ones((16,), jnp.float32)
    sl = pl.ds(s * DEG_PER_TILE, DEG_PER_TILE)
    pltpu.sync_copy(zero_v, deg_acc.at[sl])
    pltpu.sync_copy(dst_hbm.at[wid], idx_v)
    plsc.subcore_barrier()

    def body(w, carry):
        pltpu.sync_copy(ones_v, deg_acc.at[idx_v.at[w, 0]], add=True)
        return carry

    lax.fori_loop(0, NWIN_D, body, 0)
    plsc.subcore_barrier()
    pltpu.sync_copy(deg_acc.at[sl], out_hbm.at[c, 0, sl])


@functools.partial(
    pl.kernel,
    out_type=jax.ShapeDtypeStruct((NC, N2, D), jnp.float32),
    mesh=_mesh,
    scratch_types=[
        pltpu.VMEM_SHARED((N2, D), jnp.float32),
        pltpu.VMEM((GROUP, 2, K), jnp.int32),
        pltpu.VMEM((GROUP, 2, K), jnp.int32),
        pltpu.VMEM((GROUP, 2, K), jnp.int32),
        pltpu.VMEM((GROUP, 2, K), jnp.int32),
        pltpu.VMEM((GROUP, K, D), jnp.float32),
        pltpu.VMEM((GROUP, K, D), jnp.float32),
        pltpu.SemaphoreType.DMA,
        pltpu.SemaphoreType.DMA,
        pltpu.SemaphoreType.DMA,
        pltpu.SemaphoreType.DMA,
        pltpu.SemaphoreType.DMA,
        pltpu.SemaphoreType.DMA,
        pltpu.SemaphoreType.DMA,
        pltpu.SemaphoreType.DMA,
        pltpu.SemaphoreType.DMA,
        pltpu.SemaphoreType.DMA,
        pltpu.SemaphoreType.DMA,
        pltpu.SemaphoreType.DMA,
    ],
)
def _agg_kernel(g_hbm, es_hbm, dum_hbm, out_hbm, acc, i0, i1, i2, i3,
                rows_a, rows_b,
                is0, is1, is2, is3, gs0, gs1, gs2, gs3, ss0, ss1, ss2, ss3):
    idx = (i0, i1, i2, i3)
    isem = (is0, is1, is2, is3)
    rows = (rows_a, rows_b)
    gsem = ((gs0, gs1), (gs2, gs3))
    ssem = ((ss0, ss1), (ss2, ss3))
    c = lax.axis_index("c")
    s = lax.axis_index("s")
    wid = c * NS + s
    rsl = pl.ds(s * ROWS_PER_TILE, ROWS_PER_TILE)
    pltpu.sync_copy(g_hbm.at[rsl], acc.at[rsl])
    plsc.subcore_barrier()

    def fire_idx(g, j):
        pltpu.async_copy(es_hbm.at[wid, pl.ds(g * GROUP, GROUP)], idx[j],
                         isem[j])

    def wait_scatter(st, b):
        pltpu.make_async_copy(dum_hbm, rows[st].at[b], ssem[st][b]).wait()

    def fire_gathers(g, j, first=False):
        pltpu.make_async_copy(es_hbm.at[wid, pl.ds(g * GROUP, GROUP)],
                              idx[j], isem[j]).wait()
        for b in range(GROUP):
            if not first:
                wait_scatter(j % 2, b)
            pltpu.async_copy(g_hbm.at[idx[j].at[b, 0]], rows[j % 2].at[b],
                             gsem[j % 2][b])

    def drain(g, j):
        for b in range(GROUP):
            pltpu.make_async_copy(g_hbm.at[idx[j].at[b, 0]],
                                  rows[j % 2].at[b], gsem[j % 2][b]).wait()
            pltpu.async_copy(rows[j % 2].at[b], acc.at[idx[j].at[b, 1]],
                             ssem[j % 2][b], add=True)

    fire_idx(0, 0)
    fire_idx(1, 1)
    fire_idx(2, 2)
    fire_gathers(0, 0, first=True)
    fire_gathers(1, 1, first=True)
    drain(0, 0)
    fire_idx(3, 3)
    for j in range(1, 4):
        fire_gathers(j + 1, (j + 1) % 4)
        drain(j, j)
        fire_idx(j + 3, (j + 3) % 4)

    def quad_body(qi, carry):
        g0 = 4 * qi
        for j in range(4):
            fire_gathers(g0 + j + 1, (j + 1) % 4)
            drain(g0 + j, j)
            fire_idx(g0 + j + 3, (j + 3) % 4)
        return carry

    lax.fori_loop(1, NGROUP // 4 - 1, quad_body, 0)
    fire_gathers(NGROUP - 3, 1)
    drain(NGROUP - 4, 0)
    fire_idx(NGROUP - 1, 3)
    fire_gathers(NGROUP - 2, 2)
    drain(NGROUP - 3, 1)
    fire_gathers(NGROUP - 1, 3)
    drain(NGROUP - 2, 2)
    drain(NGROUP - 1, 3)
    for st in range(2):
        for b in range(GROUP):
            wait_scatter(st, b)
    plsc.subcore_barrier()
    pltpu.sync_copy(acc.at[rsl], out_hbm.at[c, rsl])


ROWS_BLK = 1000
ROWS_BLK2 = 1024


def _mm1_body(x_ref, w_ref, d0_ref, d1_ref, g_ref, dinv_ref):
    dinv = lax.rsqrt(d0_ref[...] + d1_ref[...] + 1.0)
    dinv_ref[...] = dinv
    g_ref[...] = jnp.dot(x_ref[...], w_ref[...],
                         preferred_element_type=jnp.float32) * dinv


def _mid_body(p0_ref, p1_ref, g_ref, dinv_ref, b_ref, w_ref, o_ref):
    h = (p0_ref[0] + p1_ref[0] - g_ref[...]) * dinv_ref[...] + b_ref[...]
    o_ref[...] = jnp.dot(h, w_ref[...],
                         preferred_element_type=jnp.float32) * dinv_ref[...]


def _fin_body(p0_ref, p1_ref, g_ref, dinv_ref, b_ref, o_ref):
    o_ref[...] = ((p0_ref[0] + p1_ref[0] - g_ref[...]) * dinv_ref[...]
                  + b_ref[...])


def _row_specs(blk):
    return (pl.BlockSpec((blk, D), lambda i: (i, 0)),
            pl.BlockSpec((blk, 1), lambda i: (i, 0)))


_w_spec = pl.BlockSpec((D, D), lambda i: (0, 0))
_b_spec = pl.BlockSpec((1, D), lambda i: (0, 0))


def kernel(x, edge_index, W1, b1, W2, b2):
    es = jnp.stack([edge_index[0].reshape(NW, NWIN, K),
                    edge_index[1].reshape(NW, NWIN, K)], axis=2)
    dstp = jnp.pad(edge_index[1].reshape(NW, EDGES_PER_TILE),
                   ((0, 0), (0, EPT_PAD - EDGES_PER_TILE)),
                   constant_values=N2 - 1).reshape(NW, NWIN_D, 1, KD)
    b1r = b1.reshape(1, D)
    b2r = b2.reshape(1, D)

    degp = _deg_kernel(dstp)
    d0 = degp[0, 0, :, None]
    d1 = degp[1, 0, :, None]

    row_spec, col_spec = _row_specs(ROWS_BLK)
    row_spec2, col_spec2 = _row_specs(ROWS_BLK2)

    g1, dinv = pl.pallas_call(
        _mm1_body,
        grid=(N // ROWS_BLK,),
        in_specs=[row_spec, _w_spec, col_spec, col_spec],
        out_specs=(row_spec, col_spec),
        out_shape=(jax.ShapeDtypeStruct((N2, D), jnp.float32),
                   jax.ShapeDtypeStruct((N2, 1), jnp.float32)),
    )(x, W1, d0, d1)

    dum = jnp.zeros((K, D), jnp.float32)
    p = _agg_kernel(g1, es, dum)

    part0_spec2 = pl.BlockSpec((1, ROWS_BLK2, D), lambda i: (0, i, 0))
    part1_spec2 = pl.BlockSpec((1, ROWS_BLK2, D), lambda i: (1, i, 0))
    g2 = pl.pallas_call(
        _mid_body,
        grid=(N2 // ROWS_BLK2,),
        in_specs=[part0_spec2, part1_spec2, row_spec2, col_spec2, _b_spec,
                  _w_spec],
        out_specs=row_spec2,
        out_shape=jax.ShapeDtypeStruct((N2, D), jnp.float32),
    )(p, p, g1, dinv, b1r, W2)

    q = _agg_kernel(g2, es, dum)

    part0_spec = pl.BlockSpec((1, ROWS_BLK, D), lambda i: (0, i, 0))
    part1_spec = pl.BlockSpec((1, ROWS_BLK, D), lambda i: (1, i, 0))
    out = pl.pallas_call(
        _fin_body,
        grid=(N // ROWS_BLK,),
        in_specs=[part0_spec, part1_spec, row_spec, col_spec, _b_spec],
        out_specs=row_spec,
        out_shape=jax.ShapeDtypeStruct((N, D), jnp.float32),
    )(q, q, g2, dinv, b2r)

    return out
